# baseline (speedup 1.0000x reference)
import functools

import jax
import jax.numpy as jnp
from jax.experimental import pallas as pl


def _proj_body(x_ref, w_ref, o_ref):
    o_ref[...] = jnp.dot(x_ref[...], w_ref[...],
                         preferred_element_type=jnp.float32)


def _layer_body(adj_ref, s_ref, b_ref, w_ref, b2_ref, o_ref, *, final):
    acc = jnp.dot(adj_ref[...], s_ref[...],
                  preferred_element_type=jnp.float32)
    h = jnp.maximum(acc + b_ref[...], 0.0)
    y = jnp.dot(h, w_ref[...], preferred_element_type=jnp.float32)
    if final:
        y = jnp.tanh(y + b2_ref[...])
    o_ref[...] = y


def _pick_bm(n):
    for bm in (256, 200, 128, 100, 80, 64, 40, 16, 8):
        if n % bm == 0:
            return bm
    return n


@functools.partial(jax.jit, static_argnames=("final",))
def _gcn_layer(adj, s, b, w, b2, final):
    n, f = adj.shape[0], s.shape[1]
    bm = _pick_bm(n)
    grid = (n // bm,)
    return pl.pallas_call(
        functools.partial(_layer_body, final=final),
        grid=grid,
        in_specs=[
            pl.BlockSpec((bm, n), lambda i: (i, 0)),
            pl.BlockSpec((n, f), lambda i: (0, 0)),
            pl.BlockSpec((1, f), lambda i: (0, 0)),
            pl.BlockSpec((f, f), lambda i: (0, 0)),
            pl.BlockSpec((1, f), lambda i: (0, 0)),
        ],
        out_specs=pl.BlockSpec((bm, f), lambda i: (i, 0)),
        out_shape=jax.ShapeDtypeStruct((n, f), jnp.float32),
    )(adj, s, b, w, b2)


@jax.jit
def _in_proj(x, w):
    n, f = x.shape
    return pl.pallas_call(
        _proj_body,
        out_shape=jax.ShapeDtypeStruct((n, w.shape[1]), jnp.float32),
    )(x, w)


def kernel(x, adj, W1, b1, W3, b3, Wf, bf):
    b1r = b1.reshape(1, -1)
    b3r = b3.reshape(1, -1)
    bfr = bf.reshape(1, -1)
    s1 = _in_proj(x, W1)
    t = _gcn_layer(adj, s1, b1r, W3, b3r, final=False)
    out = _gcn_layer(adj, t, b3r, Wf, bfr, final=True)
    return out
